# Initial kernel scaffold; baseline (speedup 1.0000x reference)
import jax
import jax.numpy as jnp
from jax.experimental import pallas as pl


def kernel(output, mask, ind, target):
    raise NotImplementedError("write your pallas kernel here")


# speedup vs baseline: 1.6023x; 1.6023x over previous
import functools

import jax
import jax.numpy as jnp
from jax import lax
from jax.experimental import pallas as pl
from jax.experimental.pallas import tpu as pltpu
from jax.experimental.pallas import tpu_sc as plsc

B, D, H, W = 16, 10, 128, 128
M = 500
HW = H * W
MP = 512
NW = 32
CHUNK = MP // 2
NV = CHUNK // 16
NG = D * CHUNK
NIDX = NG // 128

_mesh = plsc.VectorSubcoreMesh(core_axis_name="c", subcore_axis_name="s")


@functools.partial(
    pl.kernel,
    out_type=jax.ShapeDtypeStruct((NW, 12, 16), jnp.float32),
    mesh=_mesh,
    scratch_types=[
        pltpu.VMEM((CHUNK,), jnp.int32),
        pltpu.VMEM((CHUNK,), jnp.float32),
        pltpu.VMEM((D, CHUNK), jnp.float32),
        pltpu.VMEM((NIDX, 128), jnp.int32),
        pltpu.VMEM((NIDX, 128), jnp.float32),
        pltpu.VMEM((12, 16), jnp.float32),
        pltpu.SemaphoreType.DMA,
    ],
)
def _sc_partials(flat_hbm, ind_hbm, mask_hbm, tgt_hbm, out_hbm,
                 iv, mv, tv, idx2, pred2, part, sem):
    wid = lax.axis_index("c") * 16 + lax.axis_index("s")
    b = wid // 2
    pltpu.sync_copy(ind_hbm.at[wid], iv)
    for i in range(NV):
        v = iv[pl.ds(i * 16, 16)]
        for d in range(D):
            p = d * CHUNK + i * 16
            idx2[p // 128, pl.ds(p % 128, 16)] = v + (b * D + d) * HW
    copies = [
        pltpu.async_copy(flat_hbm.at[idx2.at[j]], pred2.at[j], sem)
        for j in range(NIDX)
    ]
    pltpu.sync_copy(mask_hbm.at[wid], mv)
    pltpu.sync_copy(tgt_hbm.at[wid], tv)
    for c in copies:
        c.wait()
    nacc = jnp.zeros((16,), jnp.float32)
    dacc = [jnp.zeros((16,), jnp.float32) for _ in range(D)]
    for i in range(NV):
        mvec = mv[pl.ds(i * 16, 16)]
        nacc = nacc + mvec
        for d in range(D):
            p = d * CHUNK + i * 16
            pv = pred2[p // 128, pl.ds(p % 128, 16)]
            t = tv[d, pl.ds(i * 16, 16)]
            dacc[d] = dacc[d] + jnp.abs(pv - t) * mvec
    for d in range(D):
        part[d, :] = dacc[d]
    part[10, :] = nacc
    part[11, :] = jnp.zeros((16,), jnp.float32)
    pltpu.sync_copy(part, out_hbm.at[wid])


def _finish(p_ref, o_ref):
    x = p_ref[...]
    s = jnp.sum(x, axis=(0, 2))
    o_ref[...] = s[:10] / (s[10] + 1e-4)


@jax.jit
def kernel(output, mask, ind, target):
    flat = output.reshape(B * D * HW)
    indp = jnp.pad(ind.astype(jnp.int32), ((0, 0), (0, MP - M)))
    indp = indp.reshape(NW, CHUNK)
    maskp = jnp.pad(mask.astype(jnp.float32), ((0, 0), (0, MP - M)))
    maskp = maskp.reshape(NW, CHUNK)
    tgtp = jnp.pad(target, ((0, 0), (0, MP - M), (0, 0)))
    tgtp = tgtp.reshape(B, 2, CHUNK, D).transpose(0, 1, 3, 2)
    tgtp = tgtp.reshape(NW, D, CHUNK)
    parts = _sc_partials(flat, indp, maskp, tgtp)
    return pl.pallas_call(
        _finish,
        out_shape=jax.ShapeDtypeStruct((10,), jnp.float32),
    )(parts)


# speedup vs baseline: 1.6026x; 1.0002x over previous
import functools

import jax
import jax.numpy as jnp
from jax import lax
from jax.experimental import pallas as pl
from jax.experimental.pallas import tpu as pltpu
from jax.experimental.pallas import tpu_sc as plsc

B, D, H, W = 16, 10, 128, 128
M = 500
HW = H * W
NW = 32
PER_W = M * B // NW
CHUNK = 256
NV = CHUNK // 16
NG = D * CHUNK
NIDX = NG // 128

_mesh = plsc.VectorSubcoreMesh(core_axis_name="c", subcore_axis_name="s")


@functools.partial(
    pl.kernel,
    out_type=jax.ShapeDtypeStruct((NW, 12, 16), jnp.float32),
    mesh=_mesh,
    scratch_types=[
        pltpu.VMEM((CHUNK,), jnp.int32),
        pltpu.VMEM((CHUNK,), jnp.int32),
        pltpu.VMEM((D * CHUNK,), jnp.float32),
        pltpu.VMEM((NIDX, 128), jnp.int32),
        pltpu.VMEM((NIDX, 128), jnp.float32),
        pltpu.VMEM((12, 16), jnp.float32),
        pltpu.SemaphoreType.DMA,
        pltpu.SemaphoreType.DMA,
    ],
)
def _sc_partials(flat_hbm, ind_hbm, mask_hbm, tgt_hbm, out_hbm,
                 iv, mv, tv, idx2, pred2, part, sem, sem2):
    wid = lax.axis_index("c") * 16 + lax.axis_index("s")
    b = wid // 2
    r = (2 * wid) % 8
    a = pl.multiple_of(PER_W * wid - r, 8)
    pltpu.sync_copy(ind_hbm.at[pl.ds(a, CHUNK)], iv)
    iota = lax.iota(jnp.int32, 16)
    zeros = jnp.zeros((16,), jnp.float32)
    for i in range(NV):
        v = iv[pl.ds(i * 16, 16)]
        for d in range(D):
            p = d * CHUNK + i * 16
            idx2[p // 128, pl.ds(p % 128, 16)] = v + (b * D + d) * HW
    copies = [
        pltpu.async_copy(flat_hbm.at[idx2.at[j]], pred2.at[j], sem)
        for j in range(NIDX)
    ]
    tcopies = [
        pltpu.async_copy(tgt_hbm.at[pl.ds(d * B * M + a, CHUNK)],
                         tv.at[pl.ds(d * CHUNK, CHUNK)], sem2)
        for d in range(D)
    ]
    pltpu.sync_copy(mask_hbm.at[pl.ds(a, CHUNK)], mv)
    for c in copies:
        c.wait()
    for c in tcopies:
        c.wait()
    nacc = zeros
    dacc = [zeros for _ in range(D)]
    for i in range(NV):
        mvec = mv[pl.ds(i * 16, 16)].astype(jnp.float32)
        if i == 0:
            mvec = jnp.where(iota >= r, mvec, 0.0)
        if i == NV - 1:
            mvec = jnp.where(iota + (NV - 1) * 16 < r + PER_W, mvec, 0.0)
        nacc = nacc + mvec
        for d in range(D):
            p = d * CHUNK + i * 16
            pv = pred2[p // 128, pl.ds(p % 128, 16)]
            t = tv[pl.ds(p, 16)]
            dacc[d] = dacc[d] + jnp.abs(pv - t) * mvec
    for d in range(D):
        part[d, :] = dacc[d]
    part[10, :] = nacc
    part[11, :] = zeros
    pltpu.sync_copy(part, out_hbm.at[wid])


def _finish(p_ref, o_ref):
    x = p_ref[...]
    s = jnp.sum(x, axis=(0, 2))
    o_ref[...] = s[:10] / (s[10] + 1e-4)


@jax.jit
def kernel(output, mask, ind, target):
    flat = output.reshape(B * D * HW)
    tgt_t = target.transpose(2, 0, 1).reshape(D * B * M)
    parts = _sc_partials(flat, ind.astype(jnp.int32).reshape(B * M),
                         mask.astype(jnp.int32).reshape(B * M),
                         tgt_t)
    return pl.pallas_call(
        _finish,
        out_shape=jax.ShapeDtypeStruct((10,), jnp.float32),
    )(parts)


# speedup vs baseline: 1.6771x; 1.0465x over previous
import functools

import jax
import jax.numpy as jnp
from jax import lax
from jax.experimental import pallas as pl
from jax.experimental.pallas import tpu as pltpu
from jax.experimental.pallas import tpu_sc as plsc

B, D, H, W = 16, 10, 128, 128
M = 500
HW = H * W
NW = 32
PER_W = M * B // NW
CHUNK = 256
NV = CHUNK // 16
NG = D * CHUNK
NIDX = NG // 128

_mesh = plsc.VectorSubcoreMesh(core_axis_name="c", subcore_axis_name="s")


@functools.partial(
    pl.kernel,
    out_type=jax.ShapeDtypeStruct((NW, 12, 16), jnp.float32),
    mesh=_mesh,
    scratch_types=[
        pltpu.VMEM((CHUNK,), jnp.float32),
        pltpu.VMEM((D * CHUNK,), jnp.float32),
        pltpu.VMEM((NIDX, 128), jnp.int32),
        pltpu.VMEM((NIDX, 128), jnp.float32),
        pltpu.VMEM((12, 16), jnp.float32),
        pltpu.SemaphoreType.DMA,
        pltpu.SemaphoreType.DMA,
    ],
)
def _sc_partials(flat_hbm, staged_hbm, out_hbm,
                 imv, tv, idx2, pred2, part, sem, sem2):
    wid = lax.axis_index("c") * 16 + lax.axis_index("s")
    b = wid // 2
    r = (2 * wid) % 8
    a = pl.multiple_of(PER_W * wid - r, 8)
    pltpu.sync_copy(staged_hbm.at[pl.ds(a, CHUNK)], imv)
    tcopies = [
        pltpu.async_copy(staged_hbm.at[pl.ds((d + 1) * B * M + a, CHUNK)],
                         tv.at[pl.ds(d * CHUNK, CHUNK)], sem2)
        for d in range(D)
    ]
    iota = lax.iota(jnp.int32, 16)
    zeros = jnp.zeros((16,), jnp.float32)
    packed = [imv[pl.ds(i * 16, 16)].astype(jnp.int32) for i in range(NV)]
    for i in range(NV):
        v = packed[i] & (HW - 1)
        for d in range(D):
            p = d * CHUNK + i * 16
            idx2[p // 128, pl.ds(p % 128, 16)] = v + (b * D + d) * HW
    copies = [
        pltpu.async_copy(flat_hbm.at[idx2.at[j]], pred2.at[j], sem)
        for j in range(NIDX)
    ]
    mvecs = []
    for i in range(NV):
        mvec = lax.shift_right_logical(packed[i], 14).astype(jnp.float32)
        if i == 0:
            mvec = jnp.where(iota >= r, mvec, 0.0)
        if i == NV - 1:
            mvec = jnp.where(iota + (NV - 1) * 16 < r + PER_W, mvec, 0.0)
        mvecs.append(mvec)
    nacc = zeros
    for i in range(NV):
        nacc = nacc + mvecs[i]
    for c in tcopies:
        c.wait()
    dacc = [zeros for _ in range(D)]
    for j in range(NIDX):
        copies[j].wait()
        d = j // 2
        for k in range(8):
            i = (j % 2) * 8 + k
            p = d * CHUNK + i * 16
            pv = pred2[j, pl.ds((p % 128), 16)]
            t = tv[pl.ds(p, 16)]
            dacc[d] = dacc[d] + jnp.abs(pv - t) * mvecs[i]
    for d in range(D):
        part[d, :] = dacc[d]
    part[10, :] = nacc
    part[11, :] = zeros
    pltpu.sync_copy(part, out_hbm.at[wid])


def _finish(p_ref, o_ref):
    x = p_ref[...]
    s = jnp.sum(x, axis=(0, 2))
    o_ref[...] = s[:10] / (s[10] + 1e-4)


@jax.jit
def kernel(output, mask, ind, target):
    flat = output.reshape(B * D * HW)
    packed = (ind.astype(jnp.int32)
              + mask.astype(jnp.int32) * HW).astype(jnp.float32)
    staged = jnp.concatenate(
        [packed.reshape(B * M),
         target.transpose(2, 0, 1).reshape(D * B * M)])
    parts = _sc_partials(flat, staged)
    return pl.pallas_call(
        _finish,
        out_shape=jax.ShapeDtypeStruct((10,), jnp.float32),
    )(parts)


# speedup vs baseline: 1.6793x; 1.0013x over previous
import functools

import jax
import jax.numpy as jnp
from jax import lax
from jax.experimental import pallas as pl
from jax.experimental.pallas import tpu as pltpu
from jax.experimental.pallas import tpu_sc as plsc

B, D, H, W = 16, 10, 128, 128
M = 500
HW = H * W
NW = 32
PER_W = M * B // NW
CHUNK = 256
NV = CHUNK // 16
NG = D * CHUNK
NIDX = NG // 128

_mesh = plsc.VectorSubcoreMesh(core_axis_name="c", subcore_axis_name="s")


@functools.partial(
    pl.kernel,
    out_type=jax.ShapeDtypeStruct((NW, 12, 16), jnp.float32),
    mesh=_mesh,
    scratch_types=[
        pltpu.VMEM((CHUNK,), jnp.float32),
        pltpu.VMEM((D * CHUNK,), jnp.float32),
        pltpu.VMEM((NV, 16), jnp.float32),
        pltpu.VMEM((NIDX, 128), jnp.int32),
        pltpu.VMEM((NIDX, 128), jnp.float32),
        pltpu.VMEM((12, 16), jnp.float32),
        pltpu.SemaphoreType.DMA,
        pltpu.SemaphoreType.DMA,
    ],
)
def _sc_partials(flat_hbm, im_hbm, tgt_hbm, out_hbm,
                 imv, tv, mbuf, idx2, pred2, part, sem, sem2):
    wid = lax.axis_index("c") * 16 + lax.axis_index("s")
    b = wid // 2
    r = (2 * wid) % 8
    a = pl.multiple_of(PER_W * wid - r, 8)
    pltpu.sync_copy(im_hbm.at[pl.ds(a, CHUNK)], imv)
    tcopies = [
        pltpu.async_copy(tgt_hbm.at[pl.ds(d * B * M + a, CHUNK)],
                         tv.at[pl.ds(d * CHUNK, CHUNK)], sem2)
        for d in range(D)
    ]
    iota = lax.iota(jnp.int32, 16)
    zeros = jnp.zeros((16,), jnp.float32)
    nacc = zeros
    for i in range(NV):
        pk = imv[pl.ds(i * 16, 16)].astype(jnp.int32)
        v = pk & (HW - 1)
        for d in range(D):
            p = d * CHUNK + i * 16
            idx2[p // 128, pl.ds(p % 128, 16)] = v + (b * D + d) * HW
        mvec = lax.shift_right_logical(pk, 14).astype(jnp.float32)
        if i == 0:
            mvec = jnp.where(iota >= r, mvec, 0.0)
        if i == NV - 1:
            mvec = jnp.where(iota + (NV - 1) * 16 < r + PER_W, mvec, 0.0)
        nacc = nacc + mvec
        mbuf[i, :] = mvec
    copies = [
        pltpu.async_copy(flat_hbm.at[idx2.at[j]], pred2.at[j], sem)
        for j in range(NIDX)
    ]
    for c in tcopies:
        c.wait()
    dacc = [zeros for _ in range(D)]
    for j in range(NIDX):
        copies[j].wait()
        d = j // 2
        for k in range(8):
            i = (j % 2) * 8 + k
            p = d * CHUNK + i * 16
            pv = pred2[j, pl.ds(16 * k, 16)]
            t = tv[pl.ds(p, 16)]
            dacc[d] = dacc[d] + jnp.abs(pv - t) * mbuf[i, :]
    for d in range(D):
        part[d, :] = dacc[d]
    part[10, :] = nacc
    part[11, :] = zeros
    pltpu.sync_copy(part, out_hbm.at[wid])


def _finish(p_ref, o_ref):
    x = p_ref[...]
    s = jnp.sum(x, axis=(0, 2))
    o_ref[...] = s[:10] / (s[10] + 1e-4)


@jax.jit
def kernel(output, mask, ind, target):
    flat = output.reshape(B * D * HW)
    packed = (ind.astype(jnp.int32)
              + mask.astype(jnp.int32) * HW).astype(jnp.float32)
    parts = _sc_partials(flat, packed.reshape(B * M),
                         target.transpose(2, 0, 1).reshape(D * B * M))
    return pl.pallas_call(
        _finish,
        out_shape=jax.ShapeDtypeStruct((10,), jnp.float32),
    )(parts)
